# baseline (speedup 1.0000x reference)
import functools

import jax
import jax.numpy as jnp
from jax import lax
from jax.experimental import pallas as pl
from jax.experimental.pallas import tpu as pltpu
from jax.experimental.pallas import tpu_sc as plsc

_NC, _NS = 2, 16
_NW = _NC * _NS
_CH = 128
_GPO = 5
_OUTER = _CH * _GPO

_EMBED_BLK = 2000
_NODE_BLK = 1000


def _f32dot(a, b):
    return jnp.dot(a, b, preferred_element_type=jnp.float32)


def _embed_body(f_ref, w1, b1, w2, b2, w3, b3, v_ref):
    h = jax.nn.gelu(_f32dot(f_ref[...], w1[...]) + b1[...])
    h = jax.nn.gelu(_f32dot(h, w2[...]) + b2[...])
    v_ref[...] = _f32dot(h, w3[...]) + b3[...]


def _edge_body(p0, p1, p2, p3, q0, q1, q2, q3, r0, r1, r2, r3, v_ref,
               w1_bd, kb1_4, kw2_bd, kb2_4, kw3_bd, kb3_4,
               ws, bs, dw1, db1, dw2, db2, dw3, db3, u_ref):
    b = v_ref.shape[0]
    b4 = b // 4
    g1cat = jnp.concatenate([p0[...], p1[...], p2[...], p3[...]], axis=1)
    acat = jnp.concatenate([q0[...], q1[...], q2[...], q3[...],
                            r0[...], r1[...], r2[...], r3[...]], axis=1)
    h4 = jax.nn.gelu(_f32dot(acat, w1_bd[...]) + kb1_4[...])
    h4 = jax.nn.gelu(_f32dot(h4, kw2_bd[...]) + kb2_4[...])
    kern4 = _f32dot(h4, kw3_bd[...]) + kb3_4[...]
    mult4 = kern4 * g1cat
    s = mult4.reshape(b4, 16, 128).sum(axis=1)
    integral = jnp.concatenate(
        [s[:, 0:32], s[:, 32:64], s[:, 64:96], s[:, 96:128]],
        axis=0) * (1.0 / 16.0)
    vt = jax.nn.gelu(_f32dot(v_ref[...], ws[...]) + bs[...] + integral)
    hd = jax.nn.gelu(_f32dot(vt, dw1[...]) + db1[...])
    hd = jax.nn.gelu(_f32dot(hd, dw2[...]) + db2[...])
    u_ref[...] = _f32dot(hd, dw3[...]) + db3[...]


def _make_sc_gather(e, dv, dx):
    assert e % _OUTER == 0
    nout = e // _OUTER
    nfull = nout // _NW
    rem = nout % _NW
    mesh = plsc.VectorSubcoreMesh(core_axis_name="c", subcore_axis_name="s")

    @functools.partial(
        pl.kernel, mesh=mesh,
        out_type=[jax.ShapeDtypeStruct((e, dv), jnp.float32),
                  jax.ShapeDtypeStruct((e, dx), jnp.float32),
                  jax.ShapeDtypeStruct((e, dx), jnp.float32)],
        scratch_types=[
            pltpu.VMEM((_OUTER,), jnp.int32),
            pltpu.VMEM((_OUTER,), jnp.int32),
            pltpu.VMEM((_OUTER, dv), jnp.float32),
            pltpu.VMEM((_OUTER, dx), jnp.float32),
            pltpu.VMEM((_OUTER, dx), jnp.float32),
            pltpu.SemaphoreType.DMA,
            pltpu.SemaphoreType.DMA,
            pltpu.SemaphoreType.DMA,
        ],
        compiler_params=pltpu.CompilerParams(use_tc_tiling_on_sc=False),
    )
    def sc_gather(v_hbm, x_hbm, idx_hbm, seg_hbm, g1_hbm, g2_hbm, g3_hbm,
                  idx_v, seg_v, vrows, xrows, srows, sem_v, sem_x, sem_s):
        c = lax.axis_index("c")
        s = lax.axis_index("s")
        wid = s * _NC + c
        count = nfull + jnp.where(wid < rem, 1, 0)
        start = nfull * wid + jnp.minimum(wid, rem)

        def body(i, carry):
            ch = start + i
            ebase = pl.multiple_of(ch * _OUTER, 128)
            pltpu.sync_copy(idx_hbm.at[pl.ds(ebase, _OUTER)], idx_v)
            pltpu.sync_copy(seg_hbm.at[pl.ds(ebase, _OUTER)], seg_v)
            copies = []
            for j in range(_GPO):
                sl = pl.ds(j * _CH, _CH)
                copies.append(pltpu.async_copy(
                    v_hbm.at[idx_v.at[sl]], vrows.at[sl], sem_v))
                copies.append(pltpu.async_copy(
                    x_hbm.at[idx_v.at[sl]], xrows.at[sl], sem_x))
                copies.append(pltpu.async_copy(
                    x_hbm.at[seg_v.at[sl]], srows.at[sl], sem_s))
            for cp in copies:
                cp.wait()
            pltpu.sync_copy(vrows, g1_hbm.at[pl.ds(ebase, _OUTER)])
            pltpu.sync_copy(xrows, g2_hbm.at[pl.ds(ebase, _OUTER)])
            pltpu.sync_copy(srows, g3_hbm.at[pl.ds(ebase, _OUTER)])
            return carry

        lax.fori_loop(0, count, body, 0)

    return sc_gather


def _whole(shape):
    return pl.BlockSpec(shape, lambda i: (0,) * len(shape))


def kernel(x, F, neighbors_index, neighbors_row_splits,
           embed_params, kernel_params, decoder_params, W_skip, b_skip):
    n, d_in = x.shape
    e = neighbors_index.shape[0]
    deg = e // n
    d_f = F.shape[1]
    (ew1, eb1), (ew2, eb2), (ew3, eb3) = embed_params
    (kw1, kb1), (kw2, kb2), (kw3, kb3) = kernel_params
    (dw1, db1), (dw2, db2), (dw3, db3) = decoder_params
    h = ew1.shape[1]
    d_emb = ew3.shape[1]
    d_out = dw3.shape[1]

    r = _EMBED_BLK
    v = pl.pallas_call(
        _embed_body,
        grid=(n // r,),
        in_specs=[
            pl.BlockSpec((r, d_f), lambda i: (i, 0)),
            _whole(ew1.shape), _whole((1, h)),
            _whole(ew2.shape), _whole((1, h)),
            _whole(ew3.shape), _whole((1, d_emb)),
        ],
        out_specs=pl.BlockSpec((r, d_emb), lambda i: (i, 0)),
        out_shape=jax.ShapeDtypeStruct((n, d_emb), jnp.float32),
        compiler_params=pltpu.CompilerParams(
            dimension_semantics=("parallel",)),
    )(F, ew1, eb1.reshape(1, h), ew2, eb2.reshape(1, h),
      ew3, eb3.reshape(1, d_emb))

    dx = 16
    b = _NODE_BLK
    nblk = n // b
    x_pad = jnp.concatenate(
        [x, jnp.zeros((n, dx - d_in), jnp.float32)], axis=1)
    seg = jnp.arange(e, dtype=jnp.int32) // deg
    g1, g2, g3 = _make_sc_gather(e, d_emb, dx)(
        v, x_pad, neighbors_index, seg)

    w1a = jnp.concatenate([kw1[:d_in], jnp.zeros((dx - d_in, h), jnp.float32)])
    w1b = jnp.concatenate([kw1[d_in:], jnp.zeros((dx - d_in, h), jnp.float32)])
    bd = jax.scipy.linalg.block_diag
    w1_bd = jnp.concatenate(
        [bd(w1a, w1a, w1a, w1a), bd(w1b, w1b, w1b, w1b)], axis=0)
    kw2_bd = bd(kw2, kw2, kw2, kw2)
    kw3_bd = bd(kw3, kw3, kw3, kw3)
    kb1_4 = jnp.tile(kb1.reshape(1, h), (1, 4))
    kb2_4 = jnp.tile(kb2.reshape(1, h), (1, 4))
    kb3_4 = jnp.tile(kb3.reshape(1, d_emb), (1, 4))
    stream_specs = [
        pl.BlockSpec((4 * b, d_emb), lambda i, t=t: (4 * i + t, 0))
        for t in range(4)
    ] + [
        pl.BlockSpec((4 * b, dx), lambda i, t=t: (4 * i + t, 0))
        for t in range(4)
    ] * 2
    u = pl.pallas_call(
        _edge_body,
        grid=(nblk,),
        in_specs=stream_specs + [
            pl.BlockSpec((b, d_emb), lambda i: (i, 0)),
            _whole(w1_bd.shape), _whole((1, 4 * h)),
            _whole(kw2_bd.shape), _whole((1, 4 * h)),
            _whole(kw3_bd.shape), _whole((1, 4 * d_emb)),
            _whole(W_skip.shape), _whole((1, d_emb)),
            _whole(dw1.shape), _whole((1, h)),
            _whole(dw2.shape), _whole((1, h)),
            _whole(dw3.shape), _whole((1, d_out)),
        ],
        out_specs=pl.BlockSpec((b, d_out), lambda i: (i, 0)),
        out_shape=jax.ShapeDtypeStruct((n, d_out), jnp.float32),
        compiler_params=pltpu.CompilerParams(
            dimension_semantics=("parallel",)),
    )(g1, g1, g1, g1, g2, g2, g2, g2, g3, g3, g3, g3, v,
      w1_bd, kb1_4, kw2_bd, kb2_4,
      kw3_bd, kb3_4,
      W_skip, b_skip.reshape(1, d_emb),
      dw1, db1.reshape(1, h), dw2, db2.reshape(1, h),
      dw3, db3.reshape(1, d_out))
    return u


# speedup vs baseline: 1.0238x; 1.0238x over previous
import functools

import jax
import jax.numpy as jnp
from jax import lax
from jax.experimental import pallas as pl
from jax.experimental.pallas import tpu as pltpu
from jax.experimental.pallas import tpu_sc as plsc

_NC, _NS = 2, 16
_NW = _NC * _NS
_CH = 128
_GPO = 10
_OUTER = _CH * _GPO

_EMBED_BLK = 2000
_NODE_BLK = 1000


def _f32dot(a, b):
    return jnp.dot(a, b, preferred_element_type=jnp.float32)


def _embed_body(f_ref, w1, b1, w2, b2, w3, b3, v_ref):
    h = jax.nn.gelu(_f32dot(f_ref[...], w1[...]) + b1[...])
    h = jax.nn.gelu(_f32dot(h, w2[...]) + b2[...])
    v_ref[...] = _f32dot(h, w3[...]) + b3[...]


def _edge_body(g1_ref, g2_ref, g3_ref, v_ref,
               w1_bd, kb1_4, kw2_bd, kb2_4, kw3_bd, kb3_4,
               ws, bs, dw1, db1, dw2, db2, dw3, db3, u_ref):
    b = v_ref.shape[0]
    b4 = b // 4
    r4 = 4 * b
    g1 = g1_ref[...]
    g2 = g2_ref[...]
    g3 = g3_ref[...]
    g1cat = jnp.concatenate(
        [g1[t * r4:(t + 1) * r4] for t in range(4)], axis=1)
    acat = jnp.concatenate(
        [g2[t * r4:(t + 1) * r4] for t in range(4)]
        + [g3[t * r4:(t + 1) * r4] for t in range(4)], axis=1)
    h4 = jax.nn.gelu(_f32dot(acat, w1_bd[...]) + kb1_4[...])
    h4 = jax.nn.gelu(_f32dot(h4, kw2_bd[...]) + kb2_4[...])
    kern4 = _f32dot(h4, kw3_bd[...]) + kb3_4[...]
    mult4 = kern4 * g1cat
    s = mult4.reshape(b4, 16, 128).sum(axis=1)
    integral = jnp.concatenate(
        [s[:, 0:32], s[:, 32:64], s[:, 64:96], s[:, 96:128]],
        axis=0) * (1.0 / 16.0)
    vt = jax.nn.gelu(_f32dot(v_ref[...], ws[...]) + bs[...] + integral)
    hd = jax.nn.gelu(_f32dot(vt, dw1[...]) + db1[...])
    hd = jax.nn.gelu(_f32dot(hd, dw2[...]) + db2[...])
    u_ref[...] = _f32dot(hd, dw3[...]) + db3[...]


def _make_sc_gather(e, dv, dx):
    assert e % _OUTER == 0
    nout = e // _OUTER
    nfull = nout // _NW
    rem = nout % _NW
    mesh = plsc.VectorSubcoreMesh(core_axis_name="c", subcore_axis_name="s")

    @functools.partial(
        pl.kernel, mesh=mesh,
        out_type=[jax.ShapeDtypeStruct((e, dv), jnp.float32),
                  jax.ShapeDtypeStruct((e, dx), jnp.float32),
                  jax.ShapeDtypeStruct((e, dx), jnp.float32)],
        scratch_types=[
            pltpu.VMEM((_OUTER,), jnp.int32),
            pltpu.VMEM((_OUTER,), jnp.int32),
            pltpu.VMEM((_OUTER, dv), jnp.float32),
            pltpu.VMEM((_OUTER, dx), jnp.float32),
            pltpu.VMEM((_OUTER, dx), jnp.float32),
            pltpu.SemaphoreType.DMA,
            pltpu.SemaphoreType.DMA,
            pltpu.SemaphoreType.DMA,
        ],
        compiler_params=pltpu.CompilerParams(use_tc_tiling_on_sc=False),
    )
    def sc_gather(v_hbm, x_hbm, idx_hbm, seg_hbm, g1_hbm, g2_hbm, g3_hbm,
                  idx_v, seg_v, vrows, xrows, srows, sem_v, sem_x, sem_s):
        c = lax.axis_index("c")
        s = lax.axis_index("s")
        wid = s * _NC + c
        count = nfull + jnp.where(wid < rem, 1, 0)
        start = nfull * wid + jnp.minimum(wid, rem)

        def body(i, carry):
            ch = start + i
            ebase = pl.multiple_of(ch * _OUTER, 128)
            pltpu.sync_copy(idx_hbm.at[pl.ds(ebase, _OUTER)], idx_v)
            pltpu.sync_copy(seg_hbm.at[pl.ds(ebase, _OUTER)], seg_v)
            copies = []
            for j in range(_GPO):
                sl = pl.ds(j * _CH, _CH)
                copies.append(pltpu.async_copy(
                    v_hbm.at[idx_v.at[sl]], vrows.at[sl], sem_v))
                copies.append(pltpu.async_copy(
                    x_hbm.at[idx_v.at[sl]], xrows.at[sl], sem_x))
                copies.append(pltpu.async_copy(
                    x_hbm.at[seg_v.at[sl]], srows.at[sl], sem_s))
            for cp in copies:
                cp.wait()
            pltpu.sync_copy(vrows, g1_hbm.at[pl.ds(ebase, _OUTER)])
            pltpu.sync_copy(xrows, g2_hbm.at[pl.ds(ebase, _OUTER)])
            pltpu.sync_copy(srows, g3_hbm.at[pl.ds(ebase, _OUTER)])
            return carry

        lax.fori_loop(0, count, body, 0)

    return sc_gather


def _whole(shape):
    return pl.BlockSpec(shape, lambda i: (0,) * len(shape))


def kernel(x, F, neighbors_index, neighbors_row_splits,
           embed_params, kernel_params, decoder_params, W_skip, b_skip):
    n, d_in = x.shape
    e = neighbors_index.shape[0]
    deg = e // n
    d_f = F.shape[1]
    (ew1, eb1), (ew2, eb2), (ew3, eb3) = embed_params
    (kw1, kb1), (kw2, kb2), (kw3, kb3) = kernel_params
    (dw1, db1), (dw2, db2), (dw3, db3) = decoder_params
    h = ew1.shape[1]
    d_emb = ew3.shape[1]
    d_out = dw3.shape[1]

    r = _EMBED_BLK
    v = pl.pallas_call(
        _embed_body,
        grid=(n // r,),
        in_specs=[
            pl.BlockSpec((r, d_f), lambda i: (i, 0)),
            _whole(ew1.shape), _whole((1, h)),
            _whole(ew2.shape), _whole((1, h)),
            _whole(ew3.shape), _whole((1, d_emb)),
        ],
        out_specs=pl.BlockSpec((r, d_emb), lambda i: (i, 0)),
        out_shape=jax.ShapeDtypeStruct((n, d_emb), jnp.float32),
        compiler_params=pltpu.CompilerParams(
            dimension_semantics=("parallel",)),
    )(F, ew1, eb1.reshape(1, h), ew2, eb2.reshape(1, h),
      ew3, eb3.reshape(1, d_emb))

    dx = 16
    b = _NODE_BLK
    nblk = n // b
    x_pad = jnp.concatenate(
        [x, jnp.zeros((n, dx - d_in), jnp.float32)], axis=1)
    seg = jnp.arange(e, dtype=jnp.int32) // deg
    g1, g2, g3 = _make_sc_gather(e, d_emb, dx)(
        v, x_pad, neighbors_index, seg)

    w1a = jnp.concatenate([kw1[:d_in], jnp.zeros((dx - d_in, h), jnp.float32)])
    w1b = jnp.concatenate([kw1[d_in:], jnp.zeros((dx - d_in, h), jnp.float32)])
    bd = jax.scipy.linalg.block_diag
    w1_bd = jnp.concatenate(
        [bd(w1a, w1a, w1a, w1a), bd(w1b, w1b, w1b, w1b)], axis=0)
    kw2_bd = bd(kw2, kw2, kw2, kw2)
    kw3_bd = bd(kw3, kw3, kw3, kw3)
    kb1_4 = jnp.tile(kb1.reshape(1, h), (1, 4))
    kb2_4 = jnp.tile(kb2.reshape(1, h), (1, 4))
    kb3_4 = jnp.tile(kb3.reshape(1, d_emb), (1, 4))
    u = pl.pallas_call(
        _edge_body,
        grid=(nblk,),
        in_specs=[
            pl.BlockSpec((deg * b, d_emb), lambda i: (i, 0)),
            pl.BlockSpec((deg * b, dx), lambda i: (i, 0)),
            pl.BlockSpec((deg * b, dx), lambda i: (i, 0)),
            pl.BlockSpec((b, d_emb), lambda i: (i, 0)),
            _whole(w1_bd.shape), _whole((1, 4 * h)),
            _whole(kw2_bd.shape), _whole((1, 4 * h)),
            _whole(kw3_bd.shape), _whole((1, 4 * d_emb)),
            _whole(W_skip.shape), _whole((1, d_emb)),
            _whole(dw1.shape), _whole((1, h)),
            _whole(dw2.shape), _whole((1, h)),
            _whole(dw3.shape), _whole((1, d_out)),
        ],
        out_specs=pl.BlockSpec((b, d_out), lambda i: (i, 0)),
        out_shape=jax.ShapeDtypeStruct((n, d_out), jnp.float32),
        compiler_params=pltpu.CompilerParams(
            dimension_semantics=("parallel",)),
    )(g1, g2, g3, v,
      w1_bd, kb1_4, kw2_bd, kb2_4,
      kw3_bd, kb3_4,
      W_skip, b_skip.reshape(1, d_emb),
      dw1, db1.reshape(1, h), dw2, db2.reshape(1, h),
      dw3, db3.reshape(1, d_out))
    return u
